# baseline (speedup 1.0000x reference)
import jax
import jax.numpy as jnp
from jax.experimental import pallas as pl
from jax.experimental.pallas import tpu as pltpu

_BM = 400
_NBUF = 3


def _copy_in(adj_hbm, buf, sems, blk):
    slot = jax.lax.rem(blk, _NBUF)
    return pltpu.make_async_copy(
        adj_hbm.at[pl.ds(blk * _BM, _BM), :], buf.at[slot], sems.at[slot])


def _sage_kernel(adj_hbm, h_ref, w_ref, out_ref, buf, sems):
    i = pl.program_id(0)
    nsteps = pl.num_programs(0)

    @pl.when(i == 0)
    def _prologue():
        for k in range(_NBUF - 1):
            _copy_in(adj_hbm, buf, sems, k).start()

    @pl.when(i + _NBUF - 1 < nsteps)
    def _prefetch():
        _copy_in(adj_hbm, buf, sems, i + _NBUF - 1).start()

    _copy_in(adj_hbm, buf, sems, i).wait()
    slot = jax.lax.rem(i, _NBUF)
    x = jnp.dot(buf[slot], h_ref[...], preferred_element_type=jnp.float32)
    out_ref[...] = jax.lax.dot_general(
        x, w_ref[...], (((1,), (1,)), ((), ())),
        preferred_element_type=jnp.float32)


def kernel(adj, h, W):
    n, _ = adj.shape
    d_in = h.shape[1]
    d_out = W.shape[0]
    grid = (n // _BM,)
    return pl.pallas_call(
        _sage_kernel,
        grid=grid,
        in_specs=[
            pl.BlockSpec(memory_space=pl.ANY),
            pl.BlockSpec((n, d_in), lambda i: (0, 0)),
            pl.BlockSpec((d_out, d_in), lambda i: (0, 0)),
        ],
        out_specs=pl.BlockSpec((_BM, d_out), lambda i: (i, 0)),
        out_shape=jax.ShapeDtypeStruct((n, d_out), jnp.float32),
        scratch_shapes=[
            pltpu.VMEM((_NBUF, _BM, n), jnp.float32),
            pltpu.SemaphoreType.DMA((_NBUF,)),
        ],
        compiler_params=pltpu.CompilerParams(
            dimension_semantics=("arbitrary",)),
    )(adj, h, W)
